# speedup vs baseline: 6.4231x; 6.4231x over previous
import functools

import jax
import jax.numpy as jnp
from jax import lax
from jax.experimental import pallas as pl
from jax.experimental.pallas import tpu as pltpu
from jax.experimental.pallas import tpu_sc as plsc

N_ATOMS = 10000
N_BONDS = 320000
N_MOLS = 64
D = 128

NC = 2
NS = 16
NW = NC * NS
PER_W = N_BONDS // NW
CHUNK = 400
N_CHUNKS = PER_W // CHUNK


def _softplus(x):
    return jnp.maximum(x, 0.0) + jnp.log1p(jnp.exp(-jnp.abs(x)))


def _project_body(atom_ref, w1b_ref, w1c_ref, pa0_ref, pa1_ref):
    a = atom_ref[...]
    pa0_ref[...] = jnp.dot(a, w1b_ref[...], preferred_element_type=jnp.float32)
    pa1_ref[...] = jnp.dot(a, w1c_ref[...], preferred_element_type=jnp.float32)


def _project(atom_feats, w1b, w1c):
    blk = 2000
    grid = N_ATOMS // blk
    return pl.pallas_call(
        _project_body,
        grid=(grid,),
        in_specs=[
            pl.BlockSpec((blk, D), lambda i: (i, 0)),
            pl.BlockSpec((D, D), lambda i: (0, 0)),
            pl.BlockSpec((D, D), lambda i: (0, 0)),
        ],
        out_specs=[
            pl.BlockSpec((blk, D), lambda i: (i, 0)),
            pl.BlockSpec((blk, D), lambda i: (i, 0)),
        ],
        out_shape=[
            jax.ShapeDtypeStruct((N_ATOMS, D), jnp.float32),
            jax.ShapeDtypeStruct((N_ATOMS, D), jnp.float32),
        ],
    )(atom_feats, w1b, w1c)


def _gather_body(pa0_hbm, pa1_hbm, i0_hbm, i1_hbm, s0_hbm, s1_hbm,
                 idx0_v, idx1_v, rows0_v, rows1_v, sem0, sem1):
    wid = lax.axis_index("s") * NC + lax.axis_index("c")
    base = wid * PER_W

    def body(k, carry):
        off = base + k * CHUNK
        pltpu.sync_copy(i0_hbm.at[pl.ds(off, CHUNK)], idx0_v)
        pltpu.sync_copy(i1_hbm.at[pl.ds(off, CHUNK)], idx1_v)
        cp0 = pltpu.async_copy(pa0_hbm.at[idx0_v], rows0_v, sem0)
        cp1 = pltpu.async_copy(pa1_hbm.at[idx1_v], rows1_v, sem1)
        cp0.wait()
        cp1.wait()
        pltpu.sync_copy(rows0_v, s0_hbm.at[pl.ds(off, CHUNK)])
        pltpu.sync_copy(rows1_v, s1_hbm.at[pl.ds(off, CHUNK)])
        return carry

    lax.fori_loop(0, N_CHUNKS, body, 0)


def _gather(pa0, pa1, i0, i1):
    mesh = plsc.VectorSubcoreMesh(
        core_axis_name="c", subcore_axis_name="s", num_cores=NC, num_subcores=NS)
    kfn = functools.partial(
        pl.kernel,
        out_type=[
            jax.ShapeDtypeStruct((N_BONDS, D), jnp.float32),
            jax.ShapeDtypeStruct((N_BONDS, D), jnp.float32),
        ],
        mesh=mesh,
        scratch_types=[
            pltpu.VMEM((CHUNK,), jnp.int32),
            pltpu.VMEM((CHUNK,), jnp.int32),
            pltpu.VMEM((CHUNK, D), jnp.float32),
            pltpu.VMEM((CHUNK, D), jnp.float32),
            pltpu.SemaphoreType.DMA,
            pltpu.SemaphoreType.DMA,
        ],
    )(_gather_body)
    return kfn(pa0, pa1, i0, i1)


def _mlp_body(mol_ref, master_ref, s0_ref, s1_ref, gf_ref, w1a_ref, w1d_ref,
              w2_ref, w3_ref, b1_ref, b2_ref, b3_ref, out_ref):
    g = jnp.dot(gf_ref[...], w1d_ref[...], preferred_element_type=jnp.float32)
    oh = (mol_ref[...] == lax.broadcasted_iota(jnp.int32, (1, N_MOLS), 1)
          ).astype(jnp.float32)
    x = jnp.dot(master_ref[...], w1a_ref[...], preferred_element_type=jnp.float32)
    x = x + s0_ref[...] + s1_ref[...] + b1_ref[...]
    x = x + jnp.dot(oh, g, preferred_element_type=jnp.float32)
    h = _softplus(x)
    h = _softplus(jnp.dot(h, w2_ref[...], preferred_element_type=jnp.float32)
                  + b2_ref[...])
    out_ref[...] = (jnp.dot(h, w3_ref[...], preferred_element_type=jnp.float32)
                    + b3_ref[...])


def _mlp(mol2d, master, s0, s1, global_feats, w1a, w1d, w2, w3, b1, b2, b3):
    blk = 2000
    grid = N_BONDS // blk
    full = lambda r, c: pl.BlockSpec((r, c), lambda i: (0, 0))
    return pl.pallas_call(
        _mlp_body,
        grid=(grid,),
        in_specs=[
            pl.BlockSpec((blk, 1), lambda i: (i, 0)),
            pl.BlockSpec((blk, D), lambda i: (i, 0)),
            pl.BlockSpec((blk, D), lambda i: (i, 0)),
            pl.BlockSpec((blk, D), lambda i: (i, 0)),
            full(N_MOLS, D),
            full(D, D),
            full(D, D),
            full(D, D),
            full(D, D),
            full(1, D),
            full(1, D),
            full(1, D),
        ],
        out_specs=pl.BlockSpec((blk, D), lambda i: (i, 0)),
        out_shape=jax.ShapeDtypeStruct((N_BONDS, D), jnp.float32),
    )(mol2d, master, s0, s1, global_feats, w1a, w1d, w2, w3, b1, b2, b3)


def kernel(master_feats, atom_feats, global_feats, bond_atom_idx, bond_mol_idx,
           W1, b1, W2, b2, W3, b3):
    w1a = W1[0:D]
    w1b = W1[D:2 * D]
    w1c = W1[2 * D:3 * D]
    w1d = W1[3 * D:4 * D]
    i0 = bond_atom_idx[:, 0].astype(jnp.int32)
    i1 = bond_atom_idx[:, 1].astype(jnp.int32)
    mol2d = bond_mol_idx.astype(jnp.int32).reshape(N_BONDS, 1)

    pa0, pa1 = _project(atom_feats, w1b, w1c)
    s0, s1 = _gather(pa0, pa1, i0, i1)
    return _mlp(mol2d, master_feats, s0, s1, global_feats, w1a, w1d,
                W2, W3, b1.reshape(1, D), b2.reshape(1, D), b3.reshape(1, D))


# speedup vs baseline: 6.5828x; 1.0249x over previous
import functools

import jax
import jax.numpy as jnp
from jax import lax
from jax.experimental import pallas as pl
from jax.experimental.pallas import tpu as pltpu
from jax.experimental.pallas import tpu_sc as plsc

N_ATOMS = 10000
N_BONDS = 320000
N_MOLS = 64
D = 128

NC = 2
NS = 16
NW = NC * NS
PER_W = N_BONDS // NW
CHUNK = 200
N_CHUNKS = PER_W // CHUNK


def _softplus(x):
    return jnp.maximum(x, 0.0) + jnp.log1p(jnp.exp(-jnp.abs(x)))


def _project_body(atom_ref, w1b_ref, w1c_ref, pa0_ref, pa1_ref):
    a = atom_ref[...]
    pa0_ref[...] = jnp.dot(a, w1b_ref[...], preferred_element_type=jnp.float32)
    pa1_ref[...] = jnp.dot(a, w1c_ref[...], preferred_element_type=jnp.float32)


def _project(atom_feats, w1b, w1c):
    blk = 2000
    grid = N_ATOMS // blk
    return pl.pallas_call(
        _project_body,
        grid=(grid,),
        in_specs=[
            pl.BlockSpec((blk, D), lambda i: (i, 0)),
            pl.BlockSpec((D, D), lambda i: (0, 0)),
            pl.BlockSpec((D, D), lambda i: (0, 0)),
        ],
        out_specs=[
            pl.BlockSpec((blk, D), lambda i: (i, 0)),
            pl.BlockSpec((blk, D), lambda i: (i, 0)),
        ],
        out_shape=[
            jax.ShapeDtypeStruct((N_ATOMS, D), jnp.float32),
            jax.ShapeDtypeStruct((N_ATOMS, D), jnp.float32),
        ],
    )(atom_feats, w1b, w1c)


def _gather_body(pa0_hbm, pa1_hbm, i0_hbm, i1_hbm, s0_hbm, s1_hbm,
                 idx0_a, idx0_b, idx1_a, idx1_b,
                 rows0_a, rows0_b, rows1_a, rows1_b,
                 sem0_a, sem0_b, sem1_a, sem1_b):
    wid = lax.axis_index("s") * NC + lax.axis_index("c")
    base = wid * PER_W
    idx0 = (idx0_a, idx0_b)
    idx1 = (idx1_a, idx1_b)
    rows0 = (rows0_a, rows0_b)
    rows1 = (rows1_a, rows1_b)
    sem0 = (sem0_a, sem0_b)
    sem1 = (sem1_a, sem1_b)

    def fire(buf, chunk):
        off = base + chunk * CHUNK
        pltpu.sync_copy(i0_hbm.at[pl.ds(off, CHUNK)], idx0[buf])
        pltpu.sync_copy(i1_hbm.at[pl.ds(off, CHUNK)], idx1[buf])
        pltpu.async_copy(pa0_hbm.at[idx0[buf]], rows0[buf], sem0[buf])
        pltpu.async_copy(pa1_hbm.at[idx1[buf]], rows1[buf], sem1[buf])

    def drain(buf, chunk):
        off = base + chunk * CHUNK
        pltpu.make_async_copy(pa0_hbm.at[idx0[buf]], rows0[buf], sem0[buf]).wait()
        pltpu.make_async_copy(pa1_hbm.at[idx1[buf]], rows1[buf], sem1[buf]).wait()
        pltpu.sync_copy(rows0[buf], s0_hbm.at[pl.ds(off, CHUNK)])
        pltpu.sync_copy(rows1[buf], s1_hbm.at[pl.ds(off, CHUNK)])

    fire(0, 0)

    @pl.loop(0, N_CHUNKS, step=2)
    def _outer(c):
        for b in range(2):
            cur = c + b
            nxt = cur + 1

            @pl.when(nxt < N_CHUNKS)
            def _():
                fire(1 - b, nxt)

            drain(b, cur)


def _gather(pa0, pa1, i0, i1):
    mesh = plsc.VectorSubcoreMesh(
        core_axis_name="c", subcore_axis_name="s", num_cores=NC, num_subcores=NS)
    kfn = functools.partial(
        pl.kernel,
        out_type=[
            jax.ShapeDtypeStruct((N_BONDS, D), jnp.float32),
            jax.ShapeDtypeStruct((N_BONDS, D), jnp.float32),
        ],
        mesh=mesh,
        scratch_types=[
            pltpu.VMEM((CHUNK,), jnp.int32),
            pltpu.VMEM((CHUNK,), jnp.int32),
            pltpu.VMEM((CHUNK,), jnp.int32),
            pltpu.VMEM((CHUNK,), jnp.int32),
            pltpu.VMEM((CHUNK, D), jnp.float32),
            pltpu.VMEM((CHUNK, D), jnp.float32),
            pltpu.VMEM((CHUNK, D), jnp.float32),
            pltpu.VMEM((CHUNK, D), jnp.float32),
            pltpu.SemaphoreType.DMA,
            pltpu.SemaphoreType.DMA,
            pltpu.SemaphoreType.DMA,
            pltpu.SemaphoreType.DMA,
        ],
    )(_gather_body)
    return kfn(pa0, pa1, i0, i1)


def _mlp_body(mol_ref, master_ref, s0_ref, s1_ref, gf_ref, w1a_ref, w1d_ref,
              w2_ref, w3_ref, b1_ref, b2_ref, b3_ref, out_ref):
    g = jnp.dot(gf_ref[...], w1d_ref[...], preferred_element_type=jnp.float32)
    oh = (mol_ref[...] == lax.broadcasted_iota(jnp.int32, (1, N_MOLS), 1)
          ).astype(jnp.float32)
    x = jnp.dot(master_ref[...], w1a_ref[...], preferred_element_type=jnp.float32)
    x = x + s0_ref[...] + s1_ref[...] + b1_ref[...]
    x = x + jnp.dot(oh, g, preferred_element_type=jnp.float32)
    h = _softplus(x)
    h = _softplus(jnp.dot(h, w2_ref[...], preferred_element_type=jnp.float32)
                  + b2_ref[...])
    out_ref[...] = (jnp.dot(h, w3_ref[...], preferred_element_type=jnp.float32)
                    + b3_ref[...])


def _mlp(mol2d, master, s0, s1, global_feats, w1a, w1d, w2, w3, b1, b2, b3):
    blk = 2000
    grid = N_BONDS // blk
    full = lambda r, c: pl.BlockSpec((r, c), lambda i: (0, 0))
    return pl.pallas_call(
        _mlp_body,
        grid=(grid,),
        in_specs=[
            pl.BlockSpec((blk, 1), lambda i: (i, 0)),
            pl.BlockSpec((blk, D), lambda i: (i, 0)),
            pl.BlockSpec((blk, D), lambda i: (i, 0)),
            pl.BlockSpec((blk, D), lambda i: (i, 0)),
            full(N_MOLS, D),
            full(D, D),
            full(D, D),
            full(D, D),
            full(D, D),
            full(1, D),
            full(1, D),
            full(1, D),
        ],
        out_specs=pl.BlockSpec((blk, D), lambda i: (i, 0)),
        out_shape=jax.ShapeDtypeStruct((N_BONDS, D), jnp.float32),
    )(mol2d, master, s0, s1, global_feats, w1a, w1d, w2, w3, b1, b2, b3)


def kernel(master_feats, atom_feats, global_feats, bond_atom_idx, bond_mol_idx,
           W1, b1, W2, b2, W3, b3):
    w1a = W1[0:D]
    w1b = W1[D:2 * D]
    w1c = W1[2 * D:3 * D]
    w1d = W1[3 * D:4 * D]
    i0 = bond_atom_idx[:, 0].astype(jnp.int32)
    i1 = bond_atom_idx[:, 1].astype(jnp.int32)
    mol2d = bond_mol_idx.astype(jnp.int32).reshape(N_BONDS, 1)

    pa0, pa1 = _project(atom_feats, w1b, w1c)
    s0, s1 = _gather(pa0, pa1, i0, i1)
    return _mlp(mol2d, master_feats, s0, s1, global_feats, w1a, w1d,
                W2, W3, b1.reshape(1, D), b2.reshape(1, D), b3.reshape(1, D))


# speedup vs baseline: 6.6438x; 1.0093x over previous
import functools

import jax
import jax.numpy as jnp
from jax import lax
from jax.experimental import pallas as pl
from jax.experimental.pallas import tpu as pltpu
from jax.experimental.pallas import tpu_sc as plsc

N_ATOMS = 10000
N_BONDS = 320000
N_MOLS = 64
D = 128

NC = 2
NS = 16
NW = NC * NS
PER_W = N_BONDS // NW
CHUNK = 200
N_CHUNKS = PER_W // CHUNK


def _softplus(x):
    return jnp.maximum(x, 0.0) + jnp.log1p(jnp.exp(-jnp.abs(x)))


def _project_body(atom_ref, w1b_ref, w1c_ref, pa0_ref, pa1_ref):
    a = atom_ref[...]
    pa0_ref[...] = jnp.dot(a, w1b_ref[...], preferred_element_type=jnp.float32)
    pa1_ref[...] = jnp.dot(a, w1c_ref[...], preferred_element_type=jnp.float32)


def _project(atom_feats, w1b, w1c):
    blk = 2000
    grid = N_ATOMS // blk
    return pl.pallas_call(
        _project_body,
        grid=(grid,),
        in_specs=[
            pl.BlockSpec((blk, D), lambda i: (i, 0)),
            pl.BlockSpec((D, D), lambda i: (0, 0)),
            pl.BlockSpec((D, D), lambda i: (0, 0)),
        ],
        out_specs=[
            pl.BlockSpec((blk, D), lambda i: (i, 0)),
            pl.BlockSpec((blk, D), lambda i: (i, 0)),
        ],
        out_shape=[
            jax.ShapeDtypeStruct((N_ATOMS, D), jnp.float32),
            jax.ShapeDtypeStruct((N_ATOMS, D), jnp.float32),
        ],
    )(atom_feats, w1b, w1c)


def _gather_body(pa0_hbm, pa1_hbm, i0_hbm, i1_hbm, s0_hbm, s1_hbm,
                 idx0_v, idx1_v,
                 rows0_a, rows0_b, rows1_a, rows1_b,
                 sem0_a, sem0_b, sem1_a, sem1_b,
                 wsem0_a, wsem0_b, wsem1_a, wsem1_b):
    wid = lax.axis_index("s") * NC + lax.axis_index("c")
    base = wid * PER_W
    rows0 = (rows0_a, rows0_b)
    rows1 = (rows1_a, rows1_b)
    sem0 = (sem0_a, sem0_b)
    sem1 = (sem1_a, sem1_b)
    wsem0 = (wsem0_a, wsem0_b)
    wsem1 = (wsem1_a, wsem1_b)

    pltpu.sync_copy(i0_hbm.at[pl.ds(base, PER_W)], idx0_v)
    pltpu.sync_copy(i1_hbm.at[pl.ds(base, PER_W)], idx1_v)

    def fire(buf, chunk):
        @pl.when(chunk >= 2)
        def _():
            pltpu.make_async_copy(
                rows0[buf], s0_hbm.at[pl.ds(base, CHUNK)], wsem0[buf]).wait()
            pltpu.make_async_copy(
                rows1[buf], s1_hbm.at[pl.ds(base, CHUNK)], wsem1[buf]).wait()
        o = chunk * CHUNK
        pltpu.async_copy(pa0_hbm.at[idx0_v.at[pl.ds(o, CHUNK)]], rows0[buf],
                         sem0[buf])
        pltpu.async_copy(pa1_hbm.at[idx1_v.at[pl.ds(o, CHUNK)]], rows1[buf],
                         sem1[buf])

    def drain(buf, chunk):
        off = base + chunk * CHUNK
        pltpu.make_async_copy(pa0_hbm.at[idx0_v.at[pl.ds(0, CHUNK)]],
                              rows0[buf], sem0[buf]).wait()
        pltpu.make_async_copy(pa1_hbm.at[idx1_v.at[pl.ds(0, CHUNK)]],
                              rows1[buf], sem1[buf]).wait()
        pltpu.async_copy(rows0[buf], s0_hbm.at[pl.ds(off, CHUNK)], wsem0[buf])
        pltpu.async_copy(rows1[buf], s1_hbm.at[pl.ds(off, CHUNK)], wsem1[buf])

    fire(0, 0)

    @pl.loop(0, N_CHUNKS, step=2)
    def _outer(c):
        for b in range(2):
            cur = c + b
            nxt = cur + 1

            @pl.when(nxt < N_CHUNKS)
            def _():
                fire(1 - b, nxt)

            drain(b, cur)

    for b in range(2):
        pltpu.make_async_copy(
            rows0[b], s0_hbm.at[pl.ds(base, CHUNK)], wsem0[b]).wait()
        pltpu.make_async_copy(
            rows1[b], s1_hbm.at[pl.ds(base, CHUNK)], wsem1[b]).wait()


def _gather(pa0, pa1, i0, i1):
    mesh = plsc.VectorSubcoreMesh(
        core_axis_name="c", subcore_axis_name="s", num_cores=NC, num_subcores=NS)
    kfn = functools.partial(
        pl.kernel,
        out_type=[
            jax.ShapeDtypeStruct((N_BONDS, D), jnp.float32),
            jax.ShapeDtypeStruct((N_BONDS, D), jnp.float32),
        ],
        mesh=mesh,
        scratch_types=[
            pltpu.VMEM((PER_W,), jnp.int32),
            pltpu.VMEM((PER_W,), jnp.int32),
            pltpu.VMEM((CHUNK, D), jnp.float32),
            pltpu.VMEM((CHUNK, D), jnp.float32),
            pltpu.VMEM((CHUNK, D), jnp.float32),
            pltpu.VMEM((CHUNK, D), jnp.float32),
            pltpu.SemaphoreType.DMA,
            pltpu.SemaphoreType.DMA,
            pltpu.SemaphoreType.DMA,
            pltpu.SemaphoreType.DMA,
            pltpu.SemaphoreType.DMA,
            pltpu.SemaphoreType.DMA,
            pltpu.SemaphoreType.DMA,
            pltpu.SemaphoreType.DMA,
        ],
    )(_gather_body)
    return kfn(pa0, pa1, i0, i1)


def _mlp_body(mol_ref, master_ref, s0_ref, s1_ref, gf_ref, w1a_ref, w1d_ref,
              w2_ref, w3_ref, b1_ref, b2_ref, b3_ref, out_ref):
    g = jnp.dot(gf_ref[...], w1d_ref[...], preferred_element_type=jnp.float32)
    oh = (mol_ref[...] == lax.broadcasted_iota(jnp.int32, (1, N_MOLS), 1)
          ).astype(jnp.float32)
    x = jnp.dot(master_ref[...], w1a_ref[...], preferred_element_type=jnp.float32)
    x = x + s0_ref[...] + s1_ref[...] + b1_ref[...]
    x = x + jnp.dot(oh, g, preferred_element_type=jnp.float32)
    h = _softplus(x)
    h = _softplus(jnp.dot(h, w2_ref[...], preferred_element_type=jnp.float32)
                  + b2_ref[...])
    out_ref[...] = (jnp.dot(h, w3_ref[...], preferred_element_type=jnp.float32)
                    + b3_ref[...])


def _mlp(mol2d, master, s0, s1, global_feats, w1a, w1d, w2, w3, b1, b2, b3):
    blk = 2000
    grid = N_BONDS // blk
    full = lambda r, c: pl.BlockSpec((r, c), lambda i: (0, 0))
    return pl.pallas_call(
        _mlp_body,
        grid=(grid,),
        in_specs=[
            pl.BlockSpec((blk, 1), lambda i: (i, 0)),
            pl.BlockSpec((blk, D), lambda i: (i, 0)),
            pl.BlockSpec((blk, D), lambda i: (i, 0)),
            pl.BlockSpec((blk, D), lambda i: (i, 0)),
            full(N_MOLS, D),
            full(D, D),
            full(D, D),
            full(D, D),
            full(D, D),
            full(1, D),
            full(1, D),
            full(1, D),
        ],
        out_specs=pl.BlockSpec((blk, D), lambda i: (i, 0)),
        out_shape=jax.ShapeDtypeStruct((N_BONDS, D), jnp.float32),
    )(mol2d, master, s0, s1, global_feats, w1a, w1d, w2, w3, b1, b2, b3)


def kernel(master_feats, atom_feats, global_feats, bond_atom_idx, bond_mol_idx,
           W1, b1, W2, b2, W3, b3):
    w1a = W1[0:D]
    w1b = W1[D:2 * D]
    w1c = W1[2 * D:3 * D]
    w1d = W1[3 * D:4 * D]
    i0 = bond_atom_idx[:, 0].astype(jnp.int32)
    i1 = bond_atom_idx[:, 1].astype(jnp.int32)
    mol2d = bond_mol_idx.astype(jnp.int32).reshape(N_BONDS, 1)

    pa0, pa1 = _project(atom_feats, w1b, w1c)
    s0, s1 = _gather(pa0, pa1, i0, i1)
    return _mlp(mol2d, master_feats, s0, s1, global_feats, w1a, w1d,
                W2, W3, b1.reshape(1, D), b2.reshape(1, D), b3.reshape(1, D))


# speedup vs baseline: 8.7234x; 1.3130x over previous
import functools

import jax
import jax.numpy as jnp
from jax import lax
from jax.experimental import pallas as pl
from jax.experimental.pallas import tpu as pltpu
from jax.experimental.pallas import tpu_sc as plsc

N_ATOMS = 10000
N_BONDS = 320000
N_MOLS = 64
D = 128

NC = 2
NS = 16
NW = NC * NS
NSEG = 1
SEG = N_BONDS // NSEG
PER_W = SEG // NW
CHUNK = 200
N_CHUNKS = PER_W // CHUNK


_LOG2E = 1.4426950408889634
_LN2 = 0.6931471805599453


def _softplus(x):
    t = jnp.exp2(jnp.abs(x) * (-_LOG2E))
    return jnp.maximum(x, 0.0) + jnp.log2(1.0 + t) * _LN2


def _project_body(atom_ref, w1b_ref, w1c_ref, pa0_ref, pa1_ref):
    a = atom_ref[...]
    pa0_ref[...] = jnp.dot(a, w1b_ref[...], preferred_element_type=jnp.float32)
    pa1_ref[...] = jnp.dot(a, w1c_ref[...], preferred_element_type=jnp.float32)


def _project(atom_feats, w1b, w1c):
    blk = 2000
    grid = N_ATOMS // blk
    return pl.pallas_call(
        _project_body,
        grid=(grid,),
        in_specs=[
            pl.BlockSpec((blk, D), lambda i: (i, 0)),
            pl.BlockSpec((D, D), lambda i: (0, 0)),
            pl.BlockSpec((D, D), lambda i: (0, 0)),
        ],
        out_specs=[
            pl.BlockSpec((blk, D), lambda i: (i, 0)),
            pl.BlockSpec((blk, D), lambda i: (i, 0)),
        ],
        out_shape=[
            jax.ShapeDtypeStruct((N_ATOMS, D), jnp.float32),
            jax.ShapeDtypeStruct((N_ATOMS, D), jnp.float32),
        ],
    )(atom_feats, w1b, w1c)


def _make_gather_body(seg_base):
  def _gather_body(pa0_hbm, pa1_hbm, i0_hbm, i1_hbm, s0_hbm,
                   idx0_v, idx1_v,
                   rows0_a, rows0_b, rows1_a, rows1_b,
                   sem0_a, sem0_b, sem1_a, sem1_b,
                   wsem0_a, wsem0_b):
    wid = lax.axis_index("s") * NC + lax.axis_index("c")
    base = wid * PER_W
    rbase = seg_base + base
    rows0 = (rows0_a, rows0_b)
    rows1 = (rows1_a, rows1_b)
    sem0 = (sem0_a, sem0_b)
    sem1 = (sem1_a, sem1_b)
    wsem0 = (wsem0_a, wsem0_b)

    pltpu.sync_copy(i0_hbm.at[pl.ds(rbase, PER_W)], idx0_v)
    pltpu.sync_copy(i1_hbm.at[pl.ds(rbase, PER_W)], idx1_v)

    def fire(buf, chunk):
        @pl.when(chunk >= 2)
        def _():
            pltpu.make_async_copy(
                rows0[buf], s0_hbm.at[pl.ds(base, CHUNK)], wsem0[buf]).wait()
        o = chunk * CHUNK
        pltpu.async_copy(pa0_hbm.at[idx0_v.at[pl.ds(o, CHUNK)]], rows0[buf],
                         sem0[buf])
        pltpu.async_copy(pa1_hbm.at[idx1_v.at[pl.ds(o, CHUNK)]], rows1[buf],
                         sem1[buf])

    def drain(buf, chunk):
        off = base + chunk * CHUNK
        pltpu.make_async_copy(pa0_hbm.at[idx0_v.at[pl.ds(0, CHUNK)]],
                              rows0[buf], sem0[buf]).wait()
        pltpu.make_async_copy(pa1_hbm.at[idx1_v.at[pl.ds(0, CHUNK)]],
                              rows1[buf], sem1[buf]).wait()

        @pl.loop(0, CHUNK)
        def _row(r):
            for j in range(D // 16):
                sl = pl.ds(j * 16, 16)
                rows0[buf][r, sl] = rows0[buf][r, sl] + rows1[buf][r, sl]

        pltpu.async_copy(rows0[buf], s0_hbm.at[pl.ds(off, CHUNK)], wsem0[buf])

    fire(0, 0)

    @pl.loop(0, N_CHUNKS, step=2)
    def _outer(c):
        for b in range(2):
            cur = c + b
            nxt = cur + 1

            @pl.when(nxt < N_CHUNKS)
            def _():
                fire(1 - b, nxt)

            drain(b, cur)

    for b in range(2):
        pltpu.make_async_copy(
            rows0[b], s0_hbm.at[pl.ds(base, CHUNK)], wsem0[b]).wait()

  return _gather_body


def _gather(pa0, pa1, i0, i1, seg_base):
    mesh = plsc.VectorSubcoreMesh(
        core_axis_name="c", subcore_axis_name="s", num_cores=NC, num_subcores=NS)
    kfn = functools.partial(
        pl.kernel,
        out_type=jax.ShapeDtypeStruct((SEG, D), jnp.float32),
        mesh=mesh,
        scratch_types=[
            pltpu.VMEM((PER_W,), jnp.int32),
            pltpu.VMEM((PER_W,), jnp.int32),
            pltpu.VMEM((CHUNK, D), jnp.float32),
            pltpu.VMEM((CHUNK, D), jnp.float32),
            pltpu.VMEM((CHUNK, D), jnp.float32),
            pltpu.VMEM((CHUNK, D), jnp.float32),
            pltpu.SemaphoreType.DMA,
            pltpu.SemaphoreType.DMA,
            pltpu.SemaphoreType.DMA,
            pltpu.SemaphoreType.DMA,
            pltpu.SemaphoreType.DMA,
            pltpu.SemaphoreType.DMA,
        ],
    )(_make_gather_body(seg_base))
    return kfn(pa0, pa1, i0, i1)


def _mlp_body(mol_ref, master_ref, s_ref, gf_ref, w1a_ref, w1d_ref,
              w2_ref, w3_ref, b1_ref, b2_ref, b3_ref, out_ref):
    g = (jnp.dot(gf_ref[...], w1d_ref[...], preferred_element_type=jnp.float32)
         + b1_ref[...])
    oh = (mol_ref[...] == lax.broadcasted_iota(jnp.int32, (1, N_MOLS), 1)
          ).astype(jnp.float32)
    x = jnp.dot(master_ref[...], w1a_ref[...], preferred_element_type=jnp.float32)
    x = x + s_ref[...]
    x = x + jnp.dot(oh, g, preferred_element_type=jnp.float32)
    h = _softplus(x)
    h = _softplus(jnp.dot(h, w2_ref[...], preferred_element_type=jnp.float32)
                  + b2_ref[...])
    out_ref[...] = (jnp.dot(h, w3_ref[...], preferred_element_type=jnp.float32)
                    + b3_ref[...])


def _mlp(mol2d, master, s, global_feats, w1a, w1d, w2, w3, b1, b2, b3,
         seg_base):
    blk = 4000
    grid = SEG // blk
    boff = seg_base // blk
    full = lambda r, c: pl.BlockSpec((r, c), lambda i: (0, 0))
    return pl.pallas_call(
        _mlp_body,
        grid=(grid,),
        in_specs=[
            pl.BlockSpec((blk, 1), lambda i: (i + boff, 0)),
            pl.BlockSpec((blk, D), lambda i: (i + boff, 0)),
            pl.BlockSpec((blk, D), lambda i: (i, 0)),
            full(N_MOLS, D),
            full(D, D),
            full(D, D),
            full(D, D),
            full(D, D),
            full(1, D),
            full(1, D),
            full(1, D),
        ],
        out_specs=pl.BlockSpec((blk, D), lambda i: (i, 0)),
        out_shape=jax.ShapeDtypeStruct((SEG, D), jnp.float32),
    )(mol2d, master, s, global_feats, w1a, w1d, w2, w3, b1, b2, b3)


def kernel(master_feats, atom_feats, global_feats, bond_atom_idx, bond_mol_idx,
           W1, b1, W2, b2, W3, b3):
    w1a = W1[0:D]
    w1b = W1[D:2 * D]
    w1c = W1[2 * D:3 * D]
    w1d = W1[3 * D:4 * D]
    i0 = bond_atom_idx[:, 0].astype(jnp.int32)
    i1 = bond_atom_idx[:, 1].astype(jnp.int32)
    mol2d = bond_mol_idx.astype(jnp.int32).reshape(N_BONDS, 1)

    pa0, pa1 = _project(atom_feats, w1b, w1c)
    outs = []
    for t in range(NSEG):
        s = _gather(pa0, pa1, i0, i1, t * SEG)
        outs.append(
            _mlp(mol2d, master_feats, s, global_feats, w1a, w1d,
                 W2, W3, b1.reshape(1, D), b2.reshape(1, D),
                 b3.reshape(1, D), t * SEG))
    return outs[0] if NSEG == 1 else jnp.concatenate(outs, axis=0)
